# baseline (speedup 1.0000x reference)
import functools

import jax
import jax.numpy as jnp
from jax import lax
from jax.experimental import pallas as pl
from jax.experimental.pallas import tpu as pltpu
from jax.experimental.pallas import tpu_sc as plsc

EXPERTS = 8
TOPK = 2
HIDDEN = 768
BT = 4096

NC = 2
NS = 16
NW = NC * NS
L = 16


def _logits_kernel(w_ref, x_ref, st_out, s_out):
    lt = lax.dot_general(
        w_ref[:], x_ref[:], (((1,), (1,)), ((), ())),
        preferred_element_type=jnp.float32)
    st_out[:, :] = jax.nn.sigmoid(lt)
    ln = lax.dot_general(
        x_ref[:], w_ref[:], (((1,), (1,)), ((), ())),
        preferred_element_type=jnp.float32)
    s_out[:, :] = jax.nn.sigmoid(ln)


def _tc_scores_t(w, x):
    tokens = x.shape[0]
    grid = (tokens // BT,)
    return pl.pallas_call(
        _logits_kernel,
        grid=grid,
        in_specs=[
            pl.BlockSpec((EXPERTS, HIDDEN), lambda i: (0, 0)),
            pl.BlockSpec((BT, HIDDEN), lambda i: (i, 0)),
        ],
        out_specs=[
            pl.BlockSpec((EXPERTS, BT), lambda i: (0, i)),
            pl.BlockSpec((BT, EXPERTS), lambda i: (i, 0)),
        ],
        out_shape=[
            jax.ShapeDtypeStruct((EXPERTS, tokens), jnp.float32),
            jax.ShapeDtypeStruct((tokens, EXPERTS), jnp.float32),
        ],
    )(w, x)


def _make_router(tokens):
    tpw = tokens // NW
    mesh = plsc.VectorSubcoreMesh(core_axis_name="c", subcore_axis_name="s")

    @functools.partial(
        pl.kernel,
        mesh=mesh,
        out_type=[
            jax.ShapeDtypeStruct((TOPK * tokens,), jnp.float32),
            jax.ShapeDtypeStruct((TOPK * tokens,), jnp.int32),
        ],
        scratch_types=[
            pltpu.VMEM((EXPERTS, tpw), jnp.float32),
            pltpu.VMEM((EXPERTS, L), jnp.float32),
            pltpu.VMEM((tpw,), jnp.float32),
            pltpu.VMEM((tpw,), jnp.float32),
            pltpu.VMEM((tpw,), jnp.int32),
            pltpu.VMEM((tpw,), jnp.int32),
        ],
    )
    def _route(st_hbm, bsp_hbm, w_hbm, i_hbm, s_v, bsp_v, w1_v, w2_v, i1_v, i2_v):
        wid = lax.axis_index("s") * NC + lax.axis_index("c")
        base = wid * tpw
        pltpu.sync_copy(st_hbm.at[:, pl.ds(base, tpw)], s_v)
        pltpu.sync_copy(bsp_hbm, bsp_v)
        bias = [bsp_v[e, :] for e in range(EXPERTS)]

        def body(j, carry):
            off = j * L
            s = [s_v[e, pl.ds(off, L)] for e in range(EXPERTS)]
            b = [s[e] + bias[e] for e in range(EXPERTS)]
            m1 = b[0]
            i1 = jnp.zeros((L,), jnp.int32)
            m2 = jnp.full((L,), -jnp.inf, jnp.float32)
            i2 = jnp.zeros((L,), jnp.int32)
            for e in range(1, EXPERTS):
                ev = jnp.full((L,), e, jnp.int32)
                gt1 = b[e] > m1
                gt2 = b[e] > m2
                m2 = jnp.where(gt1, m1, jnp.where(gt2, b[e], m2))
                i2 = jnp.where(gt1, i1, jnp.where(gt2, ev, i2))
                m1 = jnp.where(gt1, b[e], m1)
                i1 = jnp.where(gt1, ev, i1)
            w1 = s[0]
            w2 = s[0]
            for e in range(1, EXPERTS):
                ev = jnp.full((L,), e, jnp.int32)
                w1 = jnp.where(i1 == ev, s[e], w1)
                w2 = jnp.where(i2 == ev, s[e], w2)
            denom = w1 + w2
            w1_v[pl.ds(off, L)] = w1 / denom
            w2_v[pl.ds(off, L)] = w2 / denom
            i1_v[pl.ds(off, L)] = i1
            i2_v[pl.ds(off, L)] = i2
            return carry

        lax.fori_loop(0, tpw // L, body, 0)
        pltpu.sync_copy(w1_v, w_hbm.at[pl.ds(base, tpw)])
        pltpu.sync_copy(w2_v, w_hbm.at[pl.ds(tokens + base, tpw)])
        pltpu.sync_copy(i1_v, i_hbm.at[pl.ds(base, tpw)])
        pltpu.sync_copy(i2_v, i_hbm.at[pl.ds(tokens + base, tpw)])

    return _route


@jax.jit
def kernel(x, expert_embeddings, gate_bias):
    tokens = x.shape[0]
    w = expert_embeddings.astype(jnp.float32)
    bias_splat = jnp.broadcast_to(
        gate_bias.astype(jnp.float32).reshape(EXPERTS, 1), (EXPERTS, L))
    s_t, scores = _tc_scores_t(w, x.astype(jnp.float32))
    w_fl, i_fl = _make_router(tokens)(s_t, bias_splat)
    weights = w_fl.reshape(TOPK, tokens).T
    indices = i_fl.reshape(TOPK, tokens).T
    return (weights, indices, scores)


# speedup vs baseline: 1.2184x; 1.2184x over previous
import functools

import jax
import jax.numpy as jnp
from jax import lax
from jax.experimental import pallas as pl
from jax.experimental.pallas import tpu as pltpu
from jax.experimental.pallas import tpu_sc as plsc

EXPERTS = 8
TOPK = 2
HIDDEN = 768
BT = 4096

NC = 2
NS = 16
NW = NC * NS
L = 16


def _logits_kernel(w_ref, x_ref, st_out):
    lt = lax.dot_general(
        w_ref[:], x_ref[:], (((1,), (1,)), ((), ())),
        preferred_element_type=jnp.float32)
    st_out[:, :] = jax.nn.sigmoid(lt)


def _tc_scores_t(w, x):
    tokens = x.shape[0]
    grid = (tokens // BT,)
    return pl.pallas_call(
        _logits_kernel,
        grid=grid,
        in_specs=[
            pl.BlockSpec((EXPERTS, HIDDEN), lambda i: (0, 0)),
            pl.BlockSpec((BT, HIDDEN), lambda i: (i, 0)),
        ],
        out_specs=pl.BlockSpec((EXPERTS, BT), lambda i: (0, i)),
        out_shape=jax.ShapeDtypeStruct((EXPERTS, tokens), jnp.float32),
    )(w, x)


def _make_router(tokens):
    tpw = tokens // NW
    mesh = plsc.VectorSubcoreMesh(core_axis_name="c", subcore_axis_name="s")

    @functools.partial(
        pl.kernel,
        mesh=mesh,
        out_type=[
            jax.ShapeDtypeStruct((TOPK * tokens,), jnp.float32),
            jax.ShapeDtypeStruct((TOPK * tokens,), jnp.int32),
        ],
        scratch_types=[
            pltpu.VMEM((EXPERTS, tpw), jnp.float32),
            pltpu.VMEM((EXPERTS, L), jnp.float32),
            pltpu.VMEM((tpw,), jnp.float32),
            pltpu.VMEM((tpw,), jnp.float32),
            pltpu.VMEM((tpw,), jnp.int32),
            pltpu.VMEM((tpw,), jnp.int32),
        ],
    )
    def _route(st_hbm, bsp_hbm, w_hbm, i_hbm, s_v, bsp_v, w1_v, w2_v, i1_v, i2_v):
        wid = lax.axis_index("s") * NC + lax.axis_index("c")
        base = wid * tpw
        pltpu.sync_copy(st_hbm.at[:, pl.ds(base, tpw)], s_v)
        pltpu.sync_copy(bsp_hbm, bsp_v)
        bias = [bsp_v[e, :] for e in range(EXPERTS)]

        def body(j, carry):
            off = j * L
            s = [s_v[e, pl.ds(off, L)] for e in range(EXPERTS)]
            b = [s[e] + bias[e] for e in range(EXPERTS)]
            m1 = b[0]
            i1 = jnp.zeros((L,), jnp.int32)
            m2 = jnp.full((L,), -jnp.inf, jnp.float32)
            i2 = jnp.zeros((L,), jnp.int32)
            for e in range(1, EXPERTS):
                ev = jnp.full((L,), e, jnp.int32)
                gt1 = b[e] > m1
                gt2 = b[e] > m2
                m2 = jnp.where(gt1, m1, jnp.where(gt2, b[e], m2))
                i2 = jnp.where(gt1, i1, jnp.where(gt2, ev, i2))
                m1 = jnp.where(gt1, b[e], m1)
                i1 = jnp.where(gt1, ev, i1)
            w1 = s[0]
            w2 = s[0]
            for e in range(1, EXPERTS):
                ev = jnp.full((L,), e, jnp.int32)
                w1 = jnp.where(i1 == ev, s[e], w1)
                w2 = jnp.where(i2 == ev, s[e], w2)
            denom = w1 + w2
            w1_v[pl.ds(off, L)] = w1 / denom
            w2_v[pl.ds(off, L)] = w2 / denom
            i1_v[pl.ds(off, L)] = i1
            i2_v[pl.ds(off, L)] = i2
            return carry

        lax.fori_loop(0, tpw // L, body, 0)
        pltpu.sync_copy(w1_v, w_hbm.at[pl.ds(base, tpw)])
        pltpu.sync_copy(w2_v, w_hbm.at[pl.ds(tokens + base, tpw)])
        pltpu.sync_copy(i1_v, i_hbm.at[pl.ds(base, tpw)])
        pltpu.sync_copy(i2_v, i_hbm.at[pl.ds(tokens + base, tpw)])

    return _route


@jax.jit
def kernel(x, expert_embeddings, gate_bias):
    tokens = x.shape[0]
    w = expert_embeddings.astype(jnp.float32)
    bias_splat = jnp.broadcast_to(
        gate_bias.astype(jnp.float32).reshape(EXPERTS, 1), (EXPERTS, L))
    s_t = _tc_scores_t(w, x.astype(jnp.float32))
    w_fl, i_fl = _make_router(tokens)(s_t, bias_splat)
    scores = s_t.T
    weights = w_fl.reshape(TOPK, tokens).T
    indices = i_fl.reshape(TOPK, tokens).T
    return (weights, indices, scores)


# speedup vs baseline: 1.2205x; 1.0017x over previous
import functools

import jax
import jax.numpy as jnp
from jax import lax
from jax.experimental import pallas as pl
from jax.experimental.pallas import tpu as pltpu
from jax.experimental.pallas import tpu_sc as plsc

EXPERTS = 8
TOPK = 2
HIDDEN = 768
BT = 4096

NC = 2
NS = 16
NW = NC * NS
L = 16


def _logits_kernel(w_ref, x_ref, st_out):
    lt = lax.dot_general(
        w_ref[:], x_ref[:], (((1,), (1,)), ((), ())),
        preferred_element_type=jnp.float32)
    st_out[:, :] = jax.nn.sigmoid(lt)


def _tc_scores_t(w, x):
    tokens = x.shape[0]
    grid = (tokens // BT,)
    return pl.pallas_call(
        _logits_kernel,
        grid=grid,
        in_specs=[
            pl.BlockSpec((EXPERTS, HIDDEN), lambda i: (0, 0)),
            pl.BlockSpec((BT, HIDDEN), lambda i: (i, 0)),
        ],
        out_specs=pl.BlockSpec((EXPERTS, BT), lambda i: (0, i)),
        out_shape=jax.ShapeDtypeStruct((EXPERTS, tokens), jnp.float32),
    )(w, x)


def _make_router(tokens):
    tpw = tokens // NW
    mesh = plsc.VectorSubcoreMesh(core_axis_name="c", subcore_axis_name="s")

    @functools.partial(
        pl.kernel,
        mesh=mesh,
        out_type=[
            jax.ShapeDtypeStruct((TOPK * tokens,), jnp.float32),
            jax.ShapeDtypeStruct((TOPK * tokens,), jnp.int32),
        ],
        scratch_types=[
            pltpu.VMEM((EXPERTS, tpw), jnp.float32),
            pltpu.VMEM((EXPERTS, L), jnp.float32),
            pltpu.VMEM((tpw,), jnp.float32),
            pltpu.VMEM((tpw,), jnp.float32),
            pltpu.VMEM((tpw,), jnp.int32),
            pltpu.VMEM((tpw,), jnp.int32),
        ],
    )
    def _route(st_hbm, bsp_hbm, w_hbm, i_hbm, s_v, bsp_v, w1_v, w2_v, i1_v, i2_v):
        wid = lax.axis_index("s") * NC + lax.axis_index("c")
        base = wid * tpw
        pltpu.sync_copy(st_hbm.at[:, pl.ds(base, tpw)], s_v)
        pltpu.sync_copy(bsp_hbm, bsp_v)
        bias = [bsp_v[e, :] for e in range(EXPERTS)]

        @plsc.parallel_loop(0, tpw // L, unroll=4)
        def body(j):
            off = j * L
            s = [s_v[e, pl.ds(off, L)] for e in range(EXPERTS)]
            b = [s[e] + bias[e] for e in range(EXPERTS)]
            m1 = b[0]
            i1 = jnp.zeros((L,), jnp.int32)
            m2 = jnp.full((L,), -jnp.inf, jnp.float32)
            i2 = jnp.zeros((L,), jnp.int32)
            for e in range(1, EXPERTS):
                ev = jnp.full((L,), e, jnp.int32)
                gt1 = b[e] > m1
                gt2 = b[e] > m2
                m2 = jnp.where(gt1, m1, jnp.where(gt2, b[e], m2))
                i2 = jnp.where(gt1, i1, jnp.where(gt2, ev, i2))
                m1 = jnp.where(gt1, b[e], m1)
                i1 = jnp.where(gt1, ev, i1)
            w1 = s[0]
            w2 = s[0]
            for e in range(1, EXPERTS):
                ev = jnp.full((L,), e, jnp.int32)
                w1 = jnp.where(i1 == ev, s[e], w1)
                w2 = jnp.where(i2 == ev, s[e], w2)
            denom = w1 + w2
            w1_v[pl.ds(off, L)] = w1 / denom
            w2_v[pl.ds(off, L)] = w2 / denom
            i1_v[pl.ds(off, L)] = i1
            i2_v[pl.ds(off, L)] = i2

        pltpu.sync_copy(w1_v, w_hbm.at[pl.ds(base, tpw)])
        pltpu.sync_copy(w2_v, w_hbm.at[pl.ds(tokens + base, tpw)])
        pltpu.sync_copy(i1_v, i_hbm.at[pl.ds(base, tpw)])
        pltpu.sync_copy(i2_v, i_hbm.at[pl.ds(tokens + base, tpw)])

    return _route


@jax.jit
def kernel(x, expert_embeddings, gate_bias):
    tokens = x.shape[0]
    w = expert_embeddings.astype(jnp.float32)
    bias_splat = jnp.broadcast_to(
        gate_bias.astype(jnp.float32).reshape(EXPERTS, 1), (EXPERTS, L))
    s_t = _tc_scores_t(w, x.astype(jnp.float32))
    w_fl, i_fl = _make_router(tokens)(s_t, bias_splat)
    scores = s_t.T
    weights = w_fl.reshape(TOPK, tokens).T
    indices = i_fl.reshape(TOPK, tokens).T
    return (weights, indices, scores)
